# baseline (speedup 1.0000x reference)
import functools

import jax
import jax.numpy as jnp
from jax import lax
from jax.experimental import pallas as pl
from jax.experimental.pallas import tpu as pltpu
from jax.experimental.pallas import tpu_sc as plsc

VOCAB = 30522
H = 1024
MAXPOS = 4096
WIDTH = 256
LEVELS = 16
B = 4
L = 4096
EPS = 1e-12

TOKENS = B * L
NC = 2
NS = 16
NW = NC * NS
PER_W = TOKENS // NW
CHUNK = 32
NCHUNK = PER_W // CHUNK


@functools.cache
def _make_sc_gather():
  @functools.partial(
    pl.kernel,
    mesh=plsc.VectorSubcoreMesh(core_axis_name="c", subcore_axis_name="s"),
    out_type=jax.ShapeDtypeStruct((TOKENS, H), jnp.float32),
    scratch_types=[
        pltpu.VMEM((PER_W,), jnp.int32),
        pltpu.VMEM((CHUNK, H), jnp.float32),
        pltpu.VMEM((CHUNK, H), jnp.float32),
        pltpu.SemaphoreType.DMA,
        pltpu.SemaphoreType.DMA,
        pltpu.SemaphoreType.DMA,
        pltpu.SemaphoreType.DMA,
    ],
  )
  def _sc_gather(idx_hbm, table_hbm, out_hbm, idx_v, rows0, rows1,
                 gsem0, gsem1, osem0, osem1):
    wid = lax.axis_index("s") * NC + lax.axis_index("c")
    base = wid * PER_W
    pltpu.sync_copy(idx_hbm.at[pl.ds(base, PER_W)], idx_v)

    bufs = (rows0, rows1)
    gsems = (gsem0, gsem1)
    osems = (osem0, osem1)

    def gather(c):
        return pltpu.make_async_copy(
            table_hbm.at[idx_v.at[pl.ds(c * CHUNK, CHUNK)]],
            bufs[c % 2],
            gsems[c % 2],
        )

    def writeout(c):
        return pltpu.make_async_copy(
            bufs[c % 2],
            out_hbm.at[pl.ds(base + c * CHUNK, CHUNK)],
            osems[c % 2],
        )

    gather(0).start()
    for c in range(NCHUNK):
        if c + 1 < NCHUNK:
            if c >= 1:
                writeout(c - 1).wait()
            gather(c + 1).start()
        gather(c).wait()
        writeout(c).start()
    writeout(NCHUNK - 2).wait()
    writeout(NCHUNK - 1).wait()

  return _sc_gather


LN_TILE = 512
NLT = L // LN_TILE


def _gelu_exact(z):
    return 0.5 * z * (1.0 + lax.erf(z * jnp.float32(0.7071067811865476)))


def _ln_body(g_ref, tt_ref, te_ref, gamma_ref, beta_ref,
             w1, b1, w2, b2, w3, b3, gain, out_ref, pe_ref):
    i = pl.program_id(0)
    b = pl.program_id(1)

    @pl.when(b == 0)
    def _():
        pos = (i * LN_TILE + lax.broadcasted_iota(jnp.int32, (LN_TILE, 1), 0)
               ).astype(jnp.float32)
        x = pos / jnp.float32(MAXPOS - 1)
        y = x
        cv = jnp.zeros_like(y)
        weight = 0.5
        for _ in range(LEVELS):
            t = jnp.floor(y * 3.0)
            cv = cv + jnp.where(t == 2.0, jnp.float32(weight), 0.0)
            y = y * 3.0 - t
            weight = weight * 0.5
        cv = jnp.clip(cv, 0.0, 1.0)

        h = _gelu_exact(cv * w1[...] + b1[...])
        h = _gelu_exact(
            lax.dot_general(h, w2[...], (((1,), (0,)), ((), ())),
                            precision=lax.Precision.HIGHEST,
                            preferred_element_type=jnp.float32) + b2[...]
        )
        pe = lax.dot_general(h, w3[...], (((1,), (0,)), ((), ())),
                             precision=lax.Precision.HIGHEST,
                             preferred_element_type=jnp.float32) + b3[...]
        pe_ref[...] = gain[...] * pe

    te = te_ref[...]
    t0 = te[0:1, :]
    td = te[1:2, :] - t0
    emb = g_ref[...] + pe_ref[...] + t0 + tt_ref[...] * td
    mean = jnp.mean(emb, axis=1, keepdims=True)
    c = emb - mean
    var = jnp.mean(c * c, axis=1, keepdims=True)
    out_ref[...] = (c / jnp.sqrt(var + EPS)) * gamma_ref[...] + beta_ref[...]


def _fused_ln(gathered, tt_f32, type_emb, gamma, beta,
              W1, b1, W2, b2, W3, b3, pos_gain):
    return pl.pallas_call(
        _ln_body,
        grid=(NLT, B),
        in_specs=[
            pl.BlockSpec((LN_TILE, H), lambda i, b: (b * NLT + i, 0)),
            pl.BlockSpec((LN_TILE, 1), lambda i, b: (b * NLT + i, 0)),
            pl.BlockSpec((2, H), lambda i, b: (0, 0)),
            pl.BlockSpec((1, H), lambda i, b: (0, 0)),
            pl.BlockSpec((1, H), lambda i, b: (0, 0)),
            pl.BlockSpec((1, WIDTH), lambda i, b: (0, 0)),
            pl.BlockSpec((1, WIDTH), lambda i, b: (0, 0)),
            pl.BlockSpec((WIDTH, WIDTH), lambda i, b: (0, 0)),
            pl.BlockSpec((1, WIDTH), lambda i, b: (0, 0)),
            pl.BlockSpec((WIDTH, H), lambda i, b: (0, 0)),
            pl.BlockSpec((1, H), lambda i, b: (0, 0)),
            pl.BlockSpec((1, 1), lambda i, b: (0, 0)),
        ],
        out_specs=pl.BlockSpec((LN_TILE, H), lambda i, b: (b * NLT + i, 0)),
        out_shape=jax.ShapeDtypeStruct((TOKENS, H), jnp.float32),
        scratch_shapes=[pltpu.VMEM((LN_TILE, H), jnp.float32)],
    )(gathered, tt_f32, type_emb, gamma.reshape(1, H), beta.reshape(1, H),
      W1, b1.reshape(1, WIDTH), W2, b2.reshape(1, WIDTH), W3,
      b3.reshape(1, H), pos_gain.reshape(1, 1))


def kernel(input_ids, token_type_ids, word_emb, type_emb, W1, b1, W2, b2,
           W3, b3, pos_gain, gamma, beta):
    idx = input_ids.reshape(TOKENS).astype(jnp.int32)
    gathered = _make_sc_gather()(idx, word_emb)
    tt = token_type_ids.reshape(TOKENS, 1).astype(jnp.float32)
    out = _fused_ln(gathered, tt, type_emb, gamma, beta,
                    W1, b1, W2, b2, W3, b3, pos_gain)
    return out.reshape(B, L, H)


# speedup vs baseline: 1.0667x; 1.0667x over previous
import functools

import jax
import jax.numpy as jnp
from jax import lax
from jax.experimental import pallas as pl
from jax.experimental.pallas import tpu as pltpu
from jax.experimental.pallas import tpu_sc as plsc

VOCAB = 30522
H = 1024
MAXPOS = 4096
WIDTH = 256
LEVELS = 16
B = 4
L = 4096
EPS = 1e-12

TOKENS = B * L
NC = 2
NS = 16
NW = NC * NS
PER_W = TOKENS // NW
CHUNK = 32
NCHUNK = PER_W // CHUNK


@functools.cache
def _make_sc_gather():
  @functools.partial(
    pl.kernel,
    mesh=plsc.VectorSubcoreMesh(core_axis_name="c", subcore_axis_name="s"),
    out_type=jax.ShapeDtypeStruct((TOKENS, H), jnp.float32),
    scratch_types=[
        pltpu.VMEM((PER_W,), jnp.int32),
        pltpu.VMEM((CHUNK, H), jnp.float32),
        pltpu.VMEM((CHUNK, H), jnp.float32),
        pltpu.SemaphoreType.DMA,
        pltpu.SemaphoreType.DMA,
        pltpu.SemaphoreType.DMA,
        pltpu.SemaphoreType.DMA,
    ],
  )
  def _sc_gather(idx_hbm, table_hbm, out_hbm, idx_v, rows0, rows1,
                 gsem0, gsem1, osem0, osem1):
    wid = lax.axis_index("s") * NC + lax.axis_index("c")
    base = wid * PER_W
    pltpu.sync_copy(idx_hbm.at[pl.ds(base, PER_W)], idx_v)

    bufs = (rows0, rows1)
    gsems = (gsem0, gsem1)
    osems = (osem0, osem1)

    def gather(c):
        return pltpu.make_async_copy(
            table_hbm.at[idx_v.at[pl.ds(c * CHUNK, CHUNK)]],
            bufs[c % 2],
            gsems[c % 2],
        )

    def writeout(c):
        return pltpu.make_async_copy(
            bufs[c % 2],
            out_hbm.at[pl.ds(base + c * CHUNK, CHUNK)],
            osems[c % 2],
        )

    gather(0).start()
    for c in range(NCHUNK):
        if c + 1 < NCHUNK:
            if c >= 1:
                writeout(c - 1).wait()
            gather(c + 1).start()
        gather(c).wait()
        writeout(c).start()
    writeout(NCHUNK - 2).wait()
    writeout(NCHUNK - 1).wait()

  return _sc_gather


LN_TILE = 1024
NLT = L // LN_TILE


def _gelu_exact(z):
    return 0.5 * z * (1.0 + lax.erf(z * jnp.float32(0.7071067811865476)))


def _ln_body(g_ref, tt_ref, te_ref, gamma_ref, beta_ref,
             w1, b1, w2, b2, w3, b3, gain, out_ref, pe_ref):
    i = pl.program_id(0)
    b = pl.program_id(1)

    @pl.when(b == 0)
    def _():
        pos = (i * LN_TILE + lax.broadcasted_iota(jnp.int32, (LN_TILE, 1), 0)
               ).astype(jnp.float32)
        x = pos / jnp.float32(MAXPOS - 1)
        y = x
        cv = jnp.zeros_like(y)
        weight = 0.5
        for _ in range(LEVELS):
            t = jnp.floor(y * 3.0)
            cv = cv + jnp.where(t == 2.0, jnp.float32(weight), 0.0)
            y = y * 3.0 - t
            weight = weight * 0.5
        cv = jnp.clip(cv, 0.0, 1.0)

        h = _gelu_exact(cv * w1[...] + b1[...])
        h = _gelu_exact(
            lax.dot_general(h, w2[...], (((1,), (0,)), ((), ())),
                            precision=lax.Precision.HIGHEST,
                            preferred_element_type=jnp.float32) + b2[...]
        )
        pe = lax.dot_general(h, w3[...], (((1,), (0,)), ((), ())),
                             precision=lax.Precision.HIGHEST,
                             preferred_element_type=jnp.float32) + b3[...]
        pe_ref[...] = gain[...] * pe

    te = te_ref[...]
    t0 = te[0:1, :]
    td = te[1:2, :] - t0
    emb = g_ref[...] + pe_ref[...] + t0 + tt_ref[...] * td
    mean = jnp.mean(emb, axis=1, keepdims=True)
    msq = jnp.mean(emb * emb, axis=1, keepdims=True)
    var = msq - mean * mean
    scale = gamma_ref[...] / jnp.sqrt(var + EPS)
    out_ref[...] = emb * scale + (beta_ref[...] - mean * scale)


def _fused_ln(gathered, tt_f32, type_emb, gamma, beta,
              W1, b1, W2, b2, W3, b3, pos_gain):
    return pl.pallas_call(
        _ln_body,
        grid=(NLT, B),
        in_specs=[
            pl.BlockSpec((LN_TILE, H), lambda i, b: (b * NLT + i, 0)),
            pl.BlockSpec((LN_TILE, 1), lambda i, b: (b * NLT + i, 0)),
            pl.BlockSpec((2, H), lambda i, b: (0, 0)),
            pl.BlockSpec((1, H), lambda i, b: (0, 0)),
            pl.BlockSpec((1, H), lambda i, b: (0, 0)),
            pl.BlockSpec((1, WIDTH), lambda i, b: (0, 0)),
            pl.BlockSpec((1, WIDTH), lambda i, b: (0, 0)),
            pl.BlockSpec((WIDTH, WIDTH), lambda i, b: (0, 0)),
            pl.BlockSpec((1, WIDTH), lambda i, b: (0, 0)),
            pl.BlockSpec((WIDTH, H), lambda i, b: (0, 0)),
            pl.BlockSpec((1, H), lambda i, b: (0, 0)),
            pl.BlockSpec((1, 1), lambda i, b: (0, 0)),
        ],
        out_specs=pl.BlockSpec((LN_TILE, H), lambda i, b: (b * NLT + i, 0)),
        out_shape=jax.ShapeDtypeStruct((TOKENS, H), jnp.float32),
        scratch_shapes=[pltpu.VMEM((LN_TILE, H), jnp.float32)],
    )(gathered, tt_f32, type_emb, gamma.reshape(1, H), beta.reshape(1, H),
      W1, b1.reshape(1, WIDTH), W2, b2.reshape(1, WIDTH), W3,
      b3.reshape(1, H), pos_gain.reshape(1, 1))


def kernel(input_ids, token_type_ids, word_emb, type_emb, W1, b1, W2, b2,
           W3, b3, pos_gain, gamma, beta):
    idx = input_ids.reshape(TOKENS).astype(jnp.int32)
    gathered = _make_sc_gather()(idx, word_emb)
    tt = token_type_ids.reshape(TOKENS, 1).astype(jnp.float32)
    out = _fused_ln(gathered, tt, type_emb, gamma, beta,
                    W1, b1, W2, b2, W3, b3, pos_gain)
    return out.reshape(B, L, H)


# speedup vs baseline: 1.2018x; 1.1266x over previous
import functools

import jax
import jax.numpy as jnp
from jax import lax
from jax.experimental import pallas as pl
from jax.experimental.pallas import tpu as pltpu
from jax.experimental.pallas import tpu_sc as plsc

VOCAB = 30522
H = 1024
MAXPOS = 4096
WIDTH = 256
LEVELS = 16
B = 4
L = 4096
EPS = 1e-12

TOKENS = B * L
NC = 2
NS = 16
NW = NC * NS
PER_W = TOKENS // NW
CHUNK = 32
NCHUNK = PER_W // CHUNK


@functools.cache
def _make_sc_gather():
  @functools.partial(
    pl.kernel,
    mesh=plsc.VectorSubcoreMesh(core_axis_name="c", subcore_axis_name="s"),
    out_type=jax.ShapeDtypeStruct((TOKENS, H), jnp.float32),
    scratch_types=[
        pltpu.VMEM((PER_W,), jnp.int32),
        pltpu.VMEM((CHUNK, H), jnp.float32),
        pltpu.VMEM((CHUNK, H), jnp.float32),
        pltpu.SemaphoreType.DMA,
        pltpu.SemaphoreType.DMA,
        pltpu.SemaphoreType.DMA,
        pltpu.SemaphoreType.DMA,
    ],
  )
  def _sc_gather(idx_hbm, table_hbm, out_hbm, idx_v, rows0, rows1,
                 gsem0, gsem1, osem0, osem1):
    wid = lax.axis_index("s") * NC + lax.axis_index("c")
    base = wid * PER_W
    pltpu.sync_copy(idx_hbm.at[pl.ds(base, PER_W)], idx_v)

    bufs = (rows0, rows1)
    gsems = (gsem0, gsem1)
    osems = (osem0, osem1)

    def gather(c):
        return pltpu.make_async_copy(
            table_hbm.at[idx_v.at[pl.ds(c * CHUNK, CHUNK)]],
            bufs[c % 2],
            gsems[c % 2],
        )

    def writeout(c):
        return pltpu.make_async_copy(
            bufs[c % 2],
            out_hbm.at[pl.ds(base + c * CHUNK, CHUNK)],
            osems[c % 2],
        )

    gather(0).start()
    for c in range(NCHUNK):
        if c + 1 < NCHUNK:
            if c >= 1:
                writeout(c - 1).wait()
            gather(c + 1).start()
        gather(c).wait()
        writeout(c).start()
    writeout(NCHUNK - 2).wait()
    writeout(NCHUNK - 1).wait()

  return _sc_gather


LN_TILE = 1024
NLT = L // LN_TILE


def _gelu_exact(z):
    return 0.5 * z * (1.0 + lax.erf(z * jnp.float32(0.7071067811865476)))


def _ln_body(g_ref, tt_ref, te_ref, gamma_ref, beta_ref,
             w1, b1, w2, b2, w3, b3, gain, out_ref, pe_ref):
    i = pl.program_id(0)
    b = pl.program_id(1)

    @pl.when(b == 0)
    def _():
        pos = (i * LN_TILE + lax.broadcasted_iota(jnp.int32, (LN_TILE, 1), 0)
               ).astype(jnp.float32)
        x = pos / jnp.float32(MAXPOS - 1)
        y = x
        cv = jnp.zeros_like(y)
        weight = 0.5
        for _ in range(LEVELS):
            t = jnp.floor(y * 3.0)
            cv = cv + jnp.where(t == 2.0, jnp.float32(weight), 0.0)
            y = y * 3.0 - t
            weight = weight * 0.5
        cv = jnp.clip(cv, 0.0, 1.0)

        h = _gelu_exact(cv * w1[...] + b1[...])
        h = _gelu_exact(
            lax.dot_general(h, w2[...], (((1,), (0,)), ((), ())),
                            precision=lax.Precision.DEFAULT,
                            preferred_element_type=jnp.float32) + b2[...]
        )
        pe = lax.dot_general(h, w3[...], (((1,), (0,)), ((), ())),
                             precision=lax.Precision.DEFAULT,
                             preferred_element_type=jnp.float32) + b3[...]
        pe_ref[...] = gain[...] * pe

    te = te_ref[...]
    t0 = te[0:1, :]
    td = te[1:2, :] - t0
    emb = g_ref[...] + pe_ref[...] + t0 + tt_ref[...] * td
    mean = jnp.mean(emb, axis=1, keepdims=True)
    msq = jnp.mean(emb * emb, axis=1, keepdims=True)
    var = msq - mean * mean
    inv = lax.rsqrt(var + EPS)
    c = (emb - mean) * inv
    out_ref[...] = c * gamma_ref[...] + beta_ref[...]


def _fused_ln(gathered, tt_f32, type_emb, gamma, beta,
              W1, b1, W2, b2, W3, b3, pos_gain):
    return pl.pallas_call(
        _ln_body,
        grid=(NLT, B),
        in_specs=[
            pl.BlockSpec((LN_TILE, H), lambda i, b: (b * NLT + i, 0)),
            pl.BlockSpec((LN_TILE, 1), lambda i, b: (b * NLT + i, 0)),
            pl.BlockSpec((2, H), lambda i, b: (0, 0)),
            pl.BlockSpec((1, H), lambda i, b: (0, 0)),
            pl.BlockSpec((1, H), lambda i, b: (0, 0)),
            pl.BlockSpec((1, WIDTH), lambda i, b: (0, 0)),
            pl.BlockSpec((1, WIDTH), lambda i, b: (0, 0)),
            pl.BlockSpec((WIDTH, WIDTH), lambda i, b: (0, 0)),
            pl.BlockSpec((1, WIDTH), lambda i, b: (0, 0)),
            pl.BlockSpec((WIDTH, H), lambda i, b: (0, 0)),
            pl.BlockSpec((1, H), lambda i, b: (0, 0)),
            pl.BlockSpec((1, 1), lambda i, b: (0, 0)),
        ],
        out_specs=pl.BlockSpec((LN_TILE, H), lambda i, b: (b * NLT + i, 0)),
        out_shape=jax.ShapeDtypeStruct((TOKENS, H), jnp.float32),
        scratch_shapes=[pltpu.VMEM((LN_TILE, H), jnp.float32)],
    )(gathered, tt_f32, type_emb, gamma.reshape(1, H), beta.reshape(1, H),
      W1, b1.reshape(1, WIDTH), W2, b2.reshape(1, WIDTH), W3,
      b3.reshape(1, H), pos_gain.reshape(1, 1))


def kernel(input_ids, token_type_ids, word_emb, type_emb, W1, b1, W2, b2,
           W3, b3, pos_gain, gamma, beta):
    idx = input_ids.reshape(TOKENS).astype(jnp.int32)
    gathered = _make_sc_gather()(idx, word_emb)
    tt = token_type_ids.reshape(TOKENS, 1).astype(jnp.float32)
    out = _fused_ln(gathered, tt, type_emb, gamma, beta,
                    W1, b1, W2, b2, W3, b3, pos_gain)
    return out.reshape(B, L, H)


# speedup vs baseline: 1.2240x; 1.0185x over previous
import functools

import jax
import jax.numpy as jnp
from jax import lax
from jax.experimental import pallas as pl
from jax.experimental.pallas import tpu as pltpu
from jax.experimental.pallas import tpu_sc as plsc

VOCAB = 30522
H = 1024
MAXPOS = 4096
WIDTH = 256
LEVELS = 16
B = 4
L = 4096
EPS = 1e-12

TOKENS = B * L
NC = 2
NS = 16
NW = NC * NS
K = 4
LC = L // K
TOK_C = B * LC
PER_W = TOK_C // NW
CHUNK = 32
NCHUNK = PER_W // CHUNK


@functools.cache
def _make_sc_gather():
  @functools.partial(
    pl.kernel,
    mesh=plsc.VectorSubcoreMesh(core_axis_name="c", subcore_axis_name="s"),
    out_type=jax.ShapeDtypeStruct((TOK_C, H), jnp.float32),
    scratch_types=[
        pltpu.VMEM((PER_W,), jnp.int32),
        pltpu.VMEM((CHUNK, H), jnp.float32),
        pltpu.VMEM((CHUNK, H), jnp.float32),
        pltpu.SemaphoreType.DMA,
        pltpu.SemaphoreType.DMA,
        pltpu.SemaphoreType.DMA,
        pltpu.SemaphoreType.DMA,
    ],
  )
  def _sc_gather(idx_hbm, table_hbm, out_hbm, idx_v, rows0, rows1,
                 gsem0, gsem1, osem0, osem1):
    wid = lax.axis_index("s") * NC + lax.axis_index("c")
    base = wid * PER_W
    pltpu.sync_copy(idx_hbm.at[pl.ds(base, PER_W)], idx_v)

    bufs = (rows0, rows1)
    gsems = (gsem0, gsem1)
    osems = (osem0, osem1)

    def gather(c):
        return pltpu.make_async_copy(
            table_hbm.at[idx_v.at[pl.ds(c * CHUNK, CHUNK)]],
            bufs[c % 2],
            gsems[c % 2],
        )

    def writeout(c):
        return pltpu.make_async_copy(
            bufs[c % 2],
            out_hbm.at[pl.ds(base + c * CHUNK, CHUNK)],
            osems[c % 2],
        )

    gather(0).start()
    for c in range(NCHUNK):
        if c + 1 < NCHUNK:
            if c >= 1:
                writeout(c - 1).wait()
            gather(c + 1).start()
        gather(c).wait()
        writeout(c).start()
    writeout(NCHUNK - 2).wait()
    writeout(NCHUNK - 1).wait()

  return _sc_gather


def _gelu_exact(z):
    return 0.5 * z * (1.0 + lax.erf(z * jnp.float32(0.7071067811865476)))


def _make_ln_body(c, has_prev):
    def body(*refs):
        if has_prev:
            (g_ref, tt_ref, te_ref, gamma_ref, beta_ref,
             w1, b1, w2, b2, w3, b3, gain, _buf, out_ref, pe_ref) = refs
        else:
            (g_ref, tt_ref, te_ref, gamma_ref, beta_ref,
             w1, b1, w2, b2, w3, b3, gain, out_ref, pe_ref) = refs
        b = pl.program_id(0)

        @pl.when(b == 0)
        def _():
            pos = (c * LC + lax.broadcasted_iota(jnp.int32, (LC, 1), 0)
                   ).astype(jnp.float32)
            x = pos / jnp.float32(MAXPOS - 1)
            y = x
            cv = jnp.zeros_like(y)
            weight = 0.5
            for _ in range(LEVELS):
                t = jnp.floor(y * 3.0)
                cv = cv + jnp.where(t == 2.0, jnp.float32(weight), 0.0)
                y = y * 3.0 - t
                weight = weight * 0.5
            cv = jnp.clip(cv, 0.0, 1.0)

            h = _gelu_exact(cv * w1[...] + b1[...])
            h = _gelu_exact(
                lax.dot_general(h, w2[...], (((1,), (0,)), ((), ())),
                                preferred_element_type=jnp.float32) + b2[...]
            )
            pe = lax.dot_general(h, w3[...], (((1,), (0,)), ((), ())),
                                 preferred_element_type=jnp.float32) + b3[...]
            pe_ref[...] = gain[...] * pe

        te = te_ref[...]
        t0 = te[0:1, :]
        td = te[1:2, :] - t0
        emb = g_ref[...] + pe_ref[...] + t0 + tt_ref[...] * td
        mean = jnp.mean(emb, axis=1, keepdims=True)
        msq = jnp.mean(emb * emb, axis=1, keepdims=True)
        var = msq - mean * mean
        inv = lax.rsqrt(var + EPS)
        cc = (emb - mean) * inv
        out_ref[...] = cc * gamma_ref[...] + beta_ref[...]

    return body


@functools.cache
def _make_ln_call(c, has_prev):
    zero = lambda b: (0, 0)
    in_specs = [
        pl.BlockSpec((LC, H), lambda b: (b, 0)),
        pl.BlockSpec((LC, 1), lambda b: (b, 0)),
        pl.BlockSpec((2, H), zero),
        pl.BlockSpec((1, H), zero),
        pl.BlockSpec((1, H), zero),
        pl.BlockSpec((1, WIDTH), zero),
        pl.BlockSpec((1, WIDTH), zero),
        pl.BlockSpec((WIDTH, WIDTH), zero),
        pl.BlockSpec((1, WIDTH), zero),
        pl.BlockSpec((WIDTH, H), zero),
        pl.BlockSpec((1, H), zero),
        pl.BlockSpec((1, 1), zero),
    ]
    kwargs = {}
    if has_prev:
        in_specs.append(pl.BlockSpec(memory_space=pl.ANY))
        kwargs['input_output_aliases'] = {12: 0}
    return pl.pallas_call(
        _make_ln_body(c, has_prev),
        grid=(B,),
        in_specs=in_specs,
        out_specs=pl.BlockSpec((LC, H), lambda b: (b * K + c, 0)),
        out_shape=jax.ShapeDtypeStruct((TOKENS, H), jnp.float32),
        scratch_shapes=[pltpu.VMEM((LC, H), jnp.float32)],
        **kwargs,
    )


def kernel(input_ids, token_type_ids, word_emb, type_emb, W1, b1, W2, b2,
           W3, b3, pos_gain, gamma, beta):
    sc_gather = _make_sc_gather()
    ids = input_ids.astype(jnp.int32).reshape(B, K, LC)
    tts = token_type_ids.astype(jnp.float32).reshape(B, K, LC)
    args = (type_emb, gamma.reshape(1, H), beta.reshape(1, H),
            W1, b1.reshape(1, WIDTH), W2, b2.reshape(1, WIDTH), W3,
            b3.reshape(1, H), pos_gain.reshape(1, 1))

    gathered = [sc_gather(ids[:, c, :].reshape(TOK_C), word_emb)
                for c in range(K)]
    out = None
    for c in range(K):
        tt_c = tts[:, c, :].reshape(TOK_C, 1)
        ln = _make_ln_call(c, out is not None)
        if out is None:
            out = ln(gathered[c], tt_c, *args)
        else:
            out = ln(gathered[c], tt_c, *args, out)
    return out.reshape(B, L, H)


# speedup vs baseline: 1.2624x; 1.0313x over previous
import functools

import jax
import jax.numpy as jnp
from jax import lax
from jax.experimental import pallas as pl
from jax.experimental.pallas import tpu as pltpu
from jax.experimental.pallas import tpu_sc as plsc

VOCAB = 30522
H = 1024
MAXPOS = 4096
WIDTH = 256
LEVELS = 16
B = 4
L = 4096
EPS = 1e-12

TOKENS = B * L
NC = 2
NS = 16
NW = NC * NS
K = 4
LC = L // K
TOK_C = B * LC
PER_W = TOK_C // NW
CHUNK = 32
NCHUNK = PER_W // CHUNK


@functools.cache
def _make_sc_gather():
  @functools.partial(
    pl.kernel,
    mesh=plsc.VectorSubcoreMesh(core_axis_name="c", subcore_axis_name="s"),
    out_type=jax.ShapeDtypeStruct((TOK_C, H), jnp.float32),
    scratch_types=[
        pltpu.VMEM((PER_W,), jnp.int32),
        pltpu.VMEM((CHUNK, H), jnp.float32),
        pltpu.VMEM((CHUNK, H), jnp.float32),
        pltpu.SemaphoreType.DMA,
        pltpu.SemaphoreType.DMA,
        pltpu.SemaphoreType.DMA,
        pltpu.SemaphoreType.DMA,
    ],
  )
  def _sc_gather(idx_hbm, table_hbm, out_hbm, idx_v, rows0, rows1,
                 gsem0, gsem1, osem0, osem1):
    wid = lax.axis_index("s") * NC + lax.axis_index("c")
    base = wid * PER_W
    pltpu.sync_copy(idx_hbm.at[pl.ds(base, PER_W)], idx_v)

    bufs = (rows0, rows1)
    gsems = (gsem0, gsem1)
    osems = (osem0, osem1)

    def gather(c):
        return pltpu.make_async_copy(
            table_hbm.at[idx_v.at[pl.ds(c * CHUNK, CHUNK)]],
            bufs[c % 2],
            gsems[c % 2],
        )

    def writeout(c):
        return pltpu.make_async_copy(
            bufs[c % 2],
            out_hbm.at[pl.ds(base + c * CHUNK, CHUNK)],
            osems[c % 2],
        )

    gather(0).start()
    for c in range(NCHUNK):
        if c + 1 < NCHUNK:
            if c >= 1:
                writeout(c - 1).wait()
            gather(c + 1).start()
        gather(c).wait()
        writeout(c).start()
    writeout(NCHUNK - 2).wait()
    writeout(NCHUNK - 1).wait()

  return _sc_gather


def _gelu_exact(z):
    return 0.5 * z * (1.0 + lax.erf(z * jnp.float32(0.7071067811865476)))


def _make_ln_body(c, has_prev):
    def body(*refs):
        if has_prev:
            (g_ref, tt_ref, te_ref, gamma_ref, beta_ref,
             w1, b1, w2, b2, w3, b3, gain, _buf, out_ref, pe_ref) = refs
        else:
            (g_ref, tt_ref, te_ref, gamma_ref, beta_ref,
             w1, b1, w2, b2, w3, b3, gain, out_ref, pe_ref) = refs
        b = pl.program_id(0)

        @pl.when(b == 0)
        def _():
            pos = (c * LC + lax.broadcasted_iota(jnp.int32, (LC, 1), 0)
                   ).astype(jnp.float32)
            x = pos / jnp.float32(MAXPOS - 1)
            y = x
            cv = jnp.zeros_like(y)
            weight = 0.5
            for _ in range(LEVELS):
                t = jnp.floor(y * 3.0)
                cv = cv + jnp.where(t == 2.0, jnp.float32(weight), 0.0)
                y = y * 3.0 - t
                weight = weight * 0.5
            cv = jnp.clip(cv, 0.0, 1.0)

            h = _gelu_exact(cv * w1[...] + b1[...])
            h = _gelu_exact(
                lax.dot_general(h, w2[...], (((1,), (0,)), ((), ())),
                                preferred_element_type=jnp.float32) + b2[...]
            )
            pe = lax.dot_general(h, w3[...], (((1,), (0,)), ((), ())),
                                 preferred_element_type=jnp.float32) + b3[...]
            pe_ref[...] = gain[...] * pe

        te = te_ref[...]
        t0 = te[0:1, :]
        td = te[1:2, :] - t0
        tsel = lax.dot_general(tt_ref[0], td, (((0,), (0,)), ((), ())),
                               preferred_element_type=jnp.float32)
        emb = g_ref[...] + pe_ref[...] + t0 + tsel
        mean = jnp.mean(emb, axis=1, keepdims=True)
        msq = jnp.mean(emb * emb, axis=1, keepdims=True)
        var = msq - mean * mean
        inv = lax.rsqrt(var + EPS)
        cc = (emb - mean) * inv
        out_ref[...] = cc * gamma_ref[...] + beta_ref[...]

    return body


@functools.cache
def _make_ln_call(c, has_prev):
    zero = lambda b: (0, 0)
    in_specs = [
        pl.BlockSpec((LC, H), lambda b: (b, 0)),
        pl.BlockSpec((1, 1, LC), lambda b: (b, 0, 0)),
        pl.BlockSpec((2, H), zero),
        pl.BlockSpec((1, H), zero),
        pl.BlockSpec((1, H), zero),
        pl.BlockSpec((1, WIDTH), zero),
        pl.BlockSpec((1, WIDTH), zero),
        pl.BlockSpec((WIDTH, WIDTH), zero),
        pl.BlockSpec((1, WIDTH), zero),
        pl.BlockSpec((WIDTH, H), zero),
        pl.BlockSpec((1, H), zero),
        pl.BlockSpec((1, 1), zero),
    ]
    kwargs = {}
    if has_prev:
        in_specs.append(pl.BlockSpec(memory_space=pl.ANY))
        kwargs['input_output_aliases'] = {12: 0}
    return pl.pallas_call(
        _make_ln_body(c, has_prev),
        grid=(B,),
        in_specs=in_specs,
        out_specs=pl.BlockSpec((LC, H), lambda b: (b * K + c, 0)),
        out_shape=jax.ShapeDtypeStruct((TOKENS, H), jnp.float32),
        scratch_shapes=[pltpu.VMEM((LC, H), jnp.float32)],
        **kwargs,
    )


def kernel(input_ids, token_type_ids, word_emb, type_emb, W1, b1, W2, b2,
           W3, b3, pos_gain, gamma, beta):
    sc_gather = _make_sc_gather()
    ids = input_ids.astype(jnp.int32).reshape(B, K, LC)
    tts = token_type_ids.astype(jnp.float32).reshape(B, K, LC)
    args = (type_emb, gamma.reshape(1, H), beta.reshape(1, H),
            W1, b1.reshape(1, WIDTH), W2, b2.reshape(1, WIDTH), W3,
            b3.reshape(1, H), pos_gain.reshape(1, 1))

    gathered = [sc_gather(ids[:, c, :].reshape(TOK_C), word_emb)
                for c in range(K)]
    out = None
    for c in range(K):
        tt_c = tts[:, c, :].reshape(B, 1, LC)
        ln = _make_ln_call(c, out is not None)
        if out is None:
            out = ln(gathered[c], tt_c, *args)
        else:
            out = ln(gathered[c], tt_c, *args, out)
    return out.reshape(B, L, H)
